# baseline (device time: 204529 ns/iter reference)
import jax
import jax.numpy as jnp
from jax import lax
from jax.experimental import pallas as pl
from jax.experimental.pallas import tpu as pltpu

B = 4
SQ = 32
SKV = 4096
H = 8
D = 128
BK = 512
NKV = SKV // BK
T = B * NKV
SCALE = D ** -0.5


def _flash_body(q_ref, k_hbm, v_hbm, o_ref, stats_ref,
                kraw, vraw, khead, vhead, acc_s, m_s, l_s, sems, hsems):
    t = pl.program_id(0)
    kv = lax.rem(t, NKV)

    def start_fetch(step):
        slot = lax.rem(step, 2)
        bb = lax.div(step, NKV)
        rows = pl.ds(lax.rem(step, NKV) * BK, BK)
        pltpu.make_async_copy(
            k_hbm.at[bb, rows, :, :], kraw.at[slot], sems.at[slot, 0]
        ).start()
        pltpu.make_async_copy(
            v_hbm.at[bb, rows, :, :], vraw.at[slot], sems.at[slot, 1]
        ).start()

    @pl.when(t == 0)
    def _():
        start_fetch(0)

    @pl.when(t + 1 < T)
    def _():
        start_fetch(t + 1)

    @pl.when(kv == 0)
    def _():
        m_s[...] = jnp.full((H * SQ, 128), -jnp.inf, jnp.float32)
        l_s[...] = jnp.zeros((H * SQ, 128), jnp.float32)
        acc_s[...] = jnp.zeros((H * SQ, D), jnp.float32)

    slot = lax.rem(t, 2)
    pltpu.make_async_copy(
        k_hbm.at[0, pl.ds(0, BK), :, :], kraw.at[slot], sems.at[slot, 0]
    ).wait()
    pltpu.make_async_copy(
        v_hbm.at[0, pl.ds(0, BK), :, :], vraw.at[slot], sems.at[slot, 1]
    ).wait()
    for hh in range(H):
        pltpu.make_async_copy(
            kraw.at[slot, :, hh, :], khead.at[hh], hsems.at[0, hh]
        ).start()
        pltpu.make_async_copy(
            vraw.at[slot, :, hh, :], vhead.at[hh], hsems.at[1, hh]
        ).start()
    for hh in range(H):
        pltpu.make_async_copy(
            kraw.at[slot, :, hh, :], khead.at[hh], hsems.at[0, hh]
        ).wait()
        pltpu.make_async_copy(
            vraw.at[slot, :, hh, :], vhead.at[hh], hsems.at[1, hh]
        ).wait()

    for hh in range(H):
        rows = pl.ds(hh * SQ, SQ)
        q = q_ref[0, :, hh, :].astype(jnp.bfloat16)
        k = khead[hh].astype(jnp.bfloat16)
        v = vhead[hh].astype(jnp.bfloat16)

        s = lax.dot_general(
            q, k, (((1,), (1,)), ((), ())),
            preferred_element_type=jnp.float32,
            precision=lax.Precision.DEFAULT,
        ) * SCALE

        m_prev = m_s[rows, 0:1]
        m_blk = jnp.max(s, axis=1, keepdims=True)
        m_new = jnp.maximum(m_prev, m_blk)
        alpha = jnp.exp(m_prev - m_new)
        p = jnp.exp(s - m_new)

        l_new = alpha * l_s[rows, 0:1] + jnp.sum(p, axis=1, keepdims=True)
        pv = lax.dot_general(
            p.astype(jnp.bfloat16), v,
            (((1,), (0,)), ((), ())),
            preferred_element_type=jnp.float32,
            precision=lax.Precision.DEFAULT,
        )
        acc_s[rows, :] = alpha * acc_s[rows, :] + pv
        m_s[rows, :] = jnp.broadcast_to(m_new, (SQ, 128))
        l_s[rows, :] = jnp.broadcast_to(l_new, (SQ, 128))

    @pl.when(kv == NKV - 1)
    def _():
        for hh in range(H):
            rows = pl.ds(hh * SQ, SQ)
            o_ref[0, :, hh, :] = acc_s[rows, :]
            stats_ref[0, :, hh, 0:1] = m_s[rows, 0:1]
            stats_ref[0, :, hh, 1:2] = l_s[rows, 0:1]


def _comm_body(o_ref, stats_ref, out_ref, recv_o, recv_stats, send_sems, recv_sems):
    my_x = lax.axis_index("x")
    my_y = lax.axis_index("y")
    nbr = (my_x, 1 - my_y)

    barrier_sem = pltpu.get_barrier_semaphore()
    pl.semaphore_signal(
        barrier_sem, inc=1, device_id=nbr, device_id_type=pl.DeviceIdType.MESH
    )
    pl.semaphore_wait(barrier_sem, 1)

    rdma_o = pltpu.make_async_remote_copy(
        src_ref=o_ref,
        dst_ref=recv_o,
        send_sem=send_sems.at[0],
        recv_sem=recv_sems.at[0],
        device_id=nbr,
        device_id_type=pl.DeviceIdType.MESH,
    )
    rdma_s = pltpu.make_async_remote_copy(
        src_ref=stats_ref,
        dst_ref=recv_stats,
        send_sem=send_sems.at[1],
        recv_sem=recv_sems.at[1],
        device_id=nbr,
        device_id_type=pl.DeviceIdType.MESH,
    )
    rdma_o.start()
    rdma_s.start()
    rdma_o.wait()
    rdma_s.wait()

    m_a = stats_ref[..., 0:1]
    l_a = stats_ref[..., 1:2]
    m_b = recv_stats[..., 0:1]
    l_b = recv_stats[..., 1:2]
    m_g = jnp.maximum(m_a, m_b)
    ea = jnp.exp(m_a - m_g)
    eb = jnp.exp(m_b - m_g)
    denom = ea * l_a + eb * l_b
    out_ref[...] = (ea * o_ref[...] + eb * recv_o[...]) / denom


def kernel(Q, K, V):
    o_un, stats = pl.pallas_call(
        _flash_body,
        grid=(T,),
        in_specs=[
            pl.BlockSpec((1, SQ, H, D), lambda t: (t // NKV, 0, 0, 0)),
            pl.BlockSpec(memory_space=pltpu.MemorySpace.HBM),
            pl.BlockSpec(memory_space=pltpu.MemorySpace.HBM),
        ],
        out_specs=[
            pl.BlockSpec((1, SQ, H, D), lambda t: (t // NKV, 0, 0, 0)),
            pl.BlockSpec((1, SQ, H, 2), lambda t: (t // NKV, 0, 0, 0)),
        ],
        out_shape=[
            jax.ShapeDtypeStruct((B, SQ, H, D), jnp.float32),
            jax.ShapeDtypeStruct((B, SQ, H, 2), jnp.float32),
        ],
        scratch_shapes=[
            pltpu.VMEM((2, BK, H, D), jnp.float32),
            pltpu.VMEM((2, BK, H, D), jnp.float32),
            pltpu.VMEM((H, BK, D), jnp.float32),
            pltpu.VMEM((H, BK, D), jnp.float32),
            pltpu.VMEM((H * SQ, D), jnp.float32),
            pltpu.VMEM((H * SQ, 128), jnp.float32),
            pltpu.VMEM((H * SQ, 128), jnp.float32),
            pltpu.SemaphoreType.DMA((2, 2)),
            pltpu.SemaphoreType.DMA((2, H)),
        ],
    )(Q, K, V)

    out = pl.pallas_call(
        _comm_body,
        in_specs=[
            pl.BlockSpec(memory_space=pltpu.VMEM),
            pl.BlockSpec(memory_space=pltpu.VMEM),
        ],
        out_specs=pl.BlockSpec(memory_space=pltpu.VMEM),
        out_shape=jax.ShapeDtypeStruct((B, SQ, H, D), jnp.float32),
        scratch_shapes=[
            pltpu.VMEM((B, SQ, H, D), jnp.float32),
            pltpu.VMEM((B, SQ, H, 2), jnp.float32),
            pltpu.SemaphoreType.DMA((2,)),
            pltpu.SemaphoreType.DMA((2,)),
        ],
        compiler_params=pltpu.CompilerParams(collective_id=0),
    )(o_un, stats)
    return out


# device time: 137626 ns/iter; 1.4861x vs baseline; 1.4861x over previous
import jax
import jax.numpy as jnp
from jax import lax
from jax.experimental import pallas as pl
from jax.experimental.pallas import tpu as pltpu

B = 4
SQ = 32
SKV = 4096
H = 8
D = 128
BK = 512
NKV = SKV // BK
SCALE = D ** -0.5


def _flash_body(q_ref, k_ref, v_ref, o_ref, stats_ref, acc_s, m_s, l_s):
    kv = pl.program_id(1)

    @pl.when(kv == 0)
    def _():
        m_s[...] = jnp.full((H * SQ, 128), -jnp.inf, jnp.float32)
        l_s[...] = jnp.zeros((H * SQ, 128), jnp.float32)
        acc_s[...] = jnp.zeros((H * SQ, D), jnp.float32)

    for hh in range(H):
        rows = pl.ds(hh * SQ, SQ)
        q = q_ref[0, :, hh, :]
        k = k_ref[0, :, hh, :]
        v = v_ref[0, :, hh, :]

        s = lax.dot_general(
            q, k, (((1,), (1,)), ((), ())), preferred_element_type=jnp.float32
        ) * SCALE

        m_prev = m_s[rows, 0:1]
        m_blk = jnp.max(s, axis=1, keepdims=True)
        m_new = jnp.maximum(m_prev, m_blk)
        alpha = jnp.exp(m_prev - m_new)
        p = jnp.exp(s - m_new)

        l_new = alpha * l_s[rows, 0:1] + jnp.sum(p, axis=1, keepdims=True)
        pv = lax.dot_general(
            p, v, (((1,), (0,)), ((), ())), preferred_element_type=jnp.float32
        )
        acc_s[rows, :] = alpha * acc_s[rows, :] + pv
        m_s[rows, :] = jnp.broadcast_to(m_new, (SQ, 128))
        l_s[rows, :] = jnp.broadcast_to(l_new, (SQ, 128))

    @pl.when(kv == NKV - 1)
    def _():
        for hh in range(H):
            rows = pl.ds(hh * SQ, SQ)
            o_ref[0, :, hh, :] = acc_s[rows, :]
            stats_ref[0, :, hh, 0:1] = m_s[rows, 0:1]
            stats_ref[0, :, hh, 1:2] = l_s[rows, 0:1]


def _comm_body(o_ref, stats_ref, out_ref, recv_o, recv_stats, send_sems, recv_sems):
    my_x = lax.axis_index("x")
    my_y = lax.axis_index("y")
    nbr = (my_x, 1 - my_y)

    barrier_sem = pltpu.get_barrier_semaphore()
    pl.semaphore_signal(
        barrier_sem, inc=1, device_id=nbr, device_id_type=pl.DeviceIdType.MESH
    )
    pl.semaphore_wait(barrier_sem, 1)

    rdma_o = pltpu.make_async_remote_copy(
        src_ref=o_ref,
        dst_ref=recv_o,
        send_sem=send_sems.at[0],
        recv_sem=recv_sems.at[0],
        device_id=nbr,
        device_id_type=pl.DeviceIdType.MESH,
    )
    rdma_s = pltpu.make_async_remote_copy(
        src_ref=stats_ref,
        dst_ref=recv_stats,
        send_sem=send_sems.at[1],
        recv_sem=recv_sems.at[1],
        device_id=nbr,
        device_id_type=pl.DeviceIdType.MESH,
    )
    rdma_o.start()
    rdma_s.start()
    rdma_o.wait()
    rdma_s.wait()

    m_a = stats_ref[..., 0:1]
    l_a = stats_ref[..., 1:2]
    m_b = recv_stats[..., 0:1]
    l_b = recv_stats[..., 1:2]
    m_g = jnp.maximum(m_a, m_b)
    ea = jnp.exp(m_a - m_g)
    eb = jnp.exp(m_b - m_g)
    denom = ea * l_a + eb * l_b
    out_ref[...] = (ea * o_ref[...] + eb * recv_o[...]) / denom


def kernel(Q, K, V):
    o_un, stats = pl.pallas_call(
        _flash_body,
        grid=(B, NKV),
        in_specs=[
            pl.BlockSpec((1, SQ, H, D), lambda b, kv: (b, 0, 0, 0)),
            pl.BlockSpec((1, BK, H, D), lambda b, kv: (b, kv, 0, 0)),
            pl.BlockSpec((1, BK, H, D), lambda b, kv: (b, kv, 0, 0)),
        ],
        out_specs=[
            pl.BlockSpec((1, SQ, H, D), lambda b, kv: (b, 0, 0, 0)),
            pl.BlockSpec((1, SQ, H, 2), lambda b, kv: (b, 0, 0, 0)),
        ],
        out_shape=[
            jax.ShapeDtypeStruct((B, SQ, H, D), jnp.float32),
            jax.ShapeDtypeStruct((B, SQ, H, 2), jnp.float32),
        ],
        scratch_shapes=[
            pltpu.VMEM((H * SQ, D), jnp.float32),
            pltpu.VMEM((H * SQ, 128), jnp.float32),
            pltpu.VMEM((H * SQ, 128), jnp.float32),
        ],
    )(Q, K, V)

    out = pl.pallas_call(
        _comm_body,
        in_specs=[
            pl.BlockSpec(memory_space=pltpu.VMEM),
            pl.BlockSpec(memory_space=pltpu.VMEM),
        ],
        out_specs=pl.BlockSpec(memory_space=pltpu.VMEM),
        out_shape=jax.ShapeDtypeStruct((B, SQ, H, D), jnp.float32),
        scratch_shapes=[
            pltpu.VMEM((B, SQ, H, D), jnp.float32),
            pltpu.VMEM((B, SQ, H, 2), jnp.float32),
            pltpu.SemaphoreType.DMA((2,)),
            pltpu.SemaphoreType.DMA((2,)),
        ],
        compiler_params=pltpu.CompilerParams(collective_id=0),
    )(o_un, stats)
    return out
